# baseline (speedup 1.0000x reference)
import functools

import jax
import jax.numpy as jnp
from jax import lax
from jax.experimental import pallas as pl
from jax.experimental.pallas import tpu as pltpu
from jax.experimental.pallas import tpu_sc as plsc

F_SPLIT = 1024
F_BLK = 512
RC = 16
NW = 32


def _tc_partial_body(x_ref, w_ref, b_ref, h_ref):
    bi = pl.program_id(0)
    fi = pl.program_id(1)
    xb = x_ref[0]
    s = (jnp.sum(xb, axis=-1) * (1.0 / 2048.0) + jnp.max(xb, axis=-1))[None, :]
    hp = jax.lax.dot_general(
        s, w_ref[...], (((1,), (1,)), ((), ())),
        preferred_element_type=jnp.float32,
    )

    @pl.when(fi == 0)
    def _init():
        h_ref[pl.ds(bi, 1), :] = hp + b_ref[...][None, :]

    @pl.when(fi > 0)
    def _accum():
        h_ref[pl.ds(bi, 1), :] = h_ref[pl.ds(bi, 1), :] + hp


def _sc_reduce_body(x_hbm, out_hbm, buf, out_v, sems):
    f_sc = 2048 - F_SPLIT
    t_rows = 4 * f_sc // NW
    nchunk = t_rows // RC
    per_b = f_sc // (NW // 4)
    wid = lax.axis_index("s") * 2 + lax.axis_index("c")
    b_id = wid // (NW // 4)
    f0 = F_SPLIT + (wid % (NW // 4)) * per_b
    row_base = b_id * 2048 + f0

    row_iota = lax.broadcasted_iota(jnp.int32, (16,), 0)

    def start(c, slot):
        src = x_hbm.at[pl.ds(row_base + c * RC, RC), :]
        pltpu.make_async_copy(src, buf.at[slot], sems.at[slot]).start()

    start(0, 0)
    for c in range(nchunk):
        slot = c % 2
        pltpu.make_async_copy(
            x_hbm.at[pl.ds(row_base + c * RC, RC), :],
            buf.at[slot], sems.at[slot],
        ).wait()
        if c + 1 < nchunk:
            start(c + 1, (c + 1) % 2)

        zeros = jnp.zeros((16,), jnp.float32)
        ninf = jnp.full((16,), -jnp.inf, jnp.float32)
        nacc = 8

        def col_block(jo, carry):
            accs = list(carry)
            for jj in range(16):
                col = jnp.zeros((16,), jnp.int32) + (jo * 16 + jj)
                v = plsc.load_gather(buf.at[slot], [row_iota, col])
                a = jj % nacc
                accs[a] = accs[a] + v
                accs[nacc + a] = jnp.maximum(accs[nacc + a], v)
            return tuple(accs)

        accs = lax.fori_loop(
            0, 2048 // 16, col_block, (zeros,) * nacc + (ninf,) * nacc
        )
        acc_s = accs[0]
        acc_m = accs[nacc]
        for a in range(1, nacc):
            acc_s = acc_s + accs[a]
            acc_m = jnp.maximum(acc_m, accs[nacc + a])
        out_v[pl.ds(c * RC, RC)] = acc_s * (1.0 / 2048.0) + acc_m

    pltpu.sync_copy(out_v, out_hbm.at[pl.ds(wid * t_rows, t_rows)])


def _finish_body(h_ref, ssc_ref, wsc_ref, gate_ref, mask_ref):
    hp = jax.lax.dot_general(
        ssc_ref[...], wsc_ref[...], (((1,), (0,)), ((), ())),
        preferred_element_type=jnp.float32,
    )
    h = h_ref[...] + hp
    h = jnp.where(h >= 0.0, h, 0.2 * h)
    iota = jax.lax.broadcasted_iota(jnp.int32, h.shape, 1)
    m1 = jnp.max(h, axis=1, keepdims=True)
    i1 = jnp.min(jnp.where(h == m1, iota, 16), axis=1, keepdims=True)
    h2 = jnp.where(iota == i1, -jnp.inf, h)
    m2 = jnp.max(h2, axis=1, keepdims=True)
    i2 = jnp.min(jnp.where(h2 == m2, iota, 16), axis=1, keepdims=True)
    sel = (iota == i1) | (iota == i2)
    mask_ref[...] = sel.astype(jnp.float32)
    d = jnp.where(sel, jnp.exp(h - m1), 0.0)
    gate_ref[...] = d / jnp.sum(d, axis=1, keepdims=True)


def kernel(x, W, b):
    B, F, C = x.shape
    E = W.shape[0]
    f_sc = F - F_SPLIT
    t_rows = B * f_sc // NW

    h_partial = pl.pallas_call(
        _tc_partial_body,
        grid=(B, F_SPLIT // F_BLK),
        in_specs=[
            pl.BlockSpec((1, F_BLK, C), lambda bi, fi: (bi, fi, 0)),
            pl.BlockSpec((E, F_BLK), lambda bi, fi: (0, fi)),
            pl.BlockSpec((E,), lambda bi, fi: (0,)),
        ],
        out_specs=pl.BlockSpec((B, E), lambda bi, fi: (0, 0)),
        out_shape=jax.ShapeDtypeStruct((B, E), jnp.float32),
    )(x, W, b)

    sc_kernel = functools.partial(
        pl.kernel,
        mesh=plsc.VectorSubcoreMesh(core_axis_name="c", subcore_axis_name="s"),
        out_type=jax.ShapeDtypeStruct((B * f_sc,), jnp.float32),
        scratch_types=[
            pltpu.VMEM((2, RC, C), jnp.float32),
            pltpu.VMEM((t_rows,), jnp.float32),
            pltpu.SemaphoreType.DMA((2,)),
        ],
        compiler_params=pltpu.CompilerParams(
            use_tc_tiling_on_sc=False, needs_layout_passes=False
        ),
    )(_sc_reduce_body)
    s_sc = sc_kernel(x.reshape(B * F, C)).reshape(B, f_sc)

    gating, mask = pl.pallas_call(
        _finish_body,
        out_shape=[
            jax.ShapeDtypeStruct((B, E), jnp.float32),
            jax.ShapeDtypeStruct((B, E), jnp.float32),
        ],
    )(h_partial, s_sc, W.T[F_SPLIT:, :])
    return gating, mask


# speedup vs baseline: 4.7411x; 4.7411x over previous
import functools

import jax
import jax.numpy as jnp
from jax import lax
from jax.experimental import pallas as pl
from jax.experimental.pallas import tpu as pltpu
from jax.experimental.pallas import tpu_sc as plsc

F_BLK = 1024


def _logits_body(x_ref, w_ref, b_ref, h_ref):
    bi = pl.program_id(0)
    fi = pl.program_id(1)
    xb = x_ref[0]
    s = (jnp.sum(xb, axis=-1) * (1.0 / 2048.0) + jnp.max(xb, axis=-1))[None, :]
    hp = jax.lax.dot_general(
        s, w_ref[...], (((1,), (1,)), ((), ())),
        preferred_element_type=jnp.float32,
    )

    @pl.when(fi == 0)
    def _init():
        h_ref[pl.ds(bi, 1), :] = hp + b_ref[...][None, :]

    @pl.when(fi > 0)
    def _accum():
        h_ref[pl.ds(bi, 1), :] = h_ref[pl.ds(bi, 1), :] + hp


def _routing_body(h_hbm, gate_hbm, mask_hbm, hv, gv, mv):
    cid = lax.axis_index("c")
    sid = lax.axis_index("s")

    @pl.when((cid == 0) & (sid < 4))
    def _route():
        pltpu.sync_copy(h_hbm.at[sid], hv)
        v = hv[...]
        v = jnp.where(v >= 0.0, v, 0.2 * v)
        iota = lax.broadcasted_iota(jnp.int32, (16,), 0)
        m1 = jnp.max(v)
        i1 = jnp.min(jnp.where(v == m1, iota, 16))
        v2 = jnp.where(iota == i1, -jnp.inf, v)
        m2 = jnp.max(v2)
        i2 = jnp.min(jnp.where(v2 == m2, iota, 16))
        sel = (iota == i1) | (iota == i2)
        mv[...] = jnp.where(sel, 1.0, 0.0)
        d = jnp.where(sel, jnp.exp(v - m1), 0.0)
        denom = jnp.zeros((16,), jnp.float32) + jnp.sum(d)
        gv[...] = d / denom
        pltpu.sync_copy(gv, gate_hbm.at[sid])
        pltpu.sync_copy(mv, mask_hbm.at[sid])


def kernel(x, W, b):
    B, F, C = x.shape
    E = W.shape[0]

    h = pl.pallas_call(
        _logits_body,
        grid=(B, F // F_BLK),
        in_specs=[
            pl.BlockSpec((1, F_BLK, C), lambda bi, fi: (bi, fi, 0)),
            pl.BlockSpec((E, F_BLK), lambda bi, fi: (0, fi)),
            pl.BlockSpec((E,), lambda bi, fi: (0,)),
        ],
        out_specs=pl.BlockSpec((B, E), lambda bi, fi: (0, 0)),
        out_shape=jax.ShapeDtypeStruct((B, E), jnp.float32),
    )(x, W, b)

    routing = functools.partial(
        pl.kernel,
        mesh=plsc.VectorSubcoreMesh(core_axis_name="c", subcore_axis_name="s"),
        out_type=[
            jax.ShapeDtypeStruct((B, E), jnp.float32),
            jax.ShapeDtypeStruct((B, E), jnp.float32),
        ],
        scratch_types=[
            pltpu.VMEM((E,), jnp.float32),
            pltpu.VMEM((E,), jnp.float32),
            pltpu.VMEM((E,), jnp.float32),
        ],
        compiler_params=pltpu.CompilerParams(needs_layout_passes=False),
    )(_routing_body)
    gating, mask = routing(h)
    return gating, mask


# speedup vs baseline: 7.9775x; 1.6826x over previous
import jax
import jax.numpy as jnp
from jax.experimental import pallas as pl
from jax.experimental.pallas import tpu as pltpu

F_BLK = 1024


def _gate_body(x_ref, w_ref, b_ref, gate_ref, mask_ref, acc_ref):
    bi = pl.program_id(0)
    fi = pl.program_id(1)
    xb = x_ref[0]
    s = (jnp.sum(xb, axis=-1) * (1.0 / 2048.0) + jnp.max(xb, axis=-1))[None, :]
    hp = jax.lax.dot_general(
        s, w_ref[...], (((1,), (1,)), ((), ())),
        preferred_element_type=jnp.float32,
    )

    @pl.when(fi == 0)
    def _init():
        acc_ref[pl.ds(bi, 1), :] = hp + b_ref[...][None, :]

    @pl.when(fi > 0)
    def _accum():
        acc_ref[pl.ds(bi, 1), :] = acc_ref[pl.ds(bi, 1), :] + hp

    last = (bi == pl.num_programs(0) - 1) & (fi == pl.num_programs(1) - 1)

    @pl.when(last)
    def _epilogue():
        h = acc_ref[...]
        h = jnp.where(h >= 0.0, h, 0.2 * h)
        iota = jax.lax.broadcasted_iota(jnp.int32, h.shape, 1)
        m1 = jnp.max(h, axis=1, keepdims=True)
        i1 = jnp.min(jnp.where(h == m1, iota, 16), axis=1, keepdims=True)
        h2 = jnp.where(iota == i1, -jnp.inf, h)
        m2 = jnp.max(h2, axis=1, keepdims=True)
        i2 = jnp.min(jnp.where(h2 == m2, iota, 16), axis=1, keepdims=True)
        sel = (iota == i1) | (iota == i2)
        mask_ref[...] = sel.astype(jnp.float32)
        d = jnp.where(sel, jnp.exp(h - m1), 0.0)
        gate_ref[...] = d / jnp.sum(d, axis=1, keepdims=True)


def kernel(x, W, b):
    B, F, C = x.shape
    E = W.shape[0]
    grid = (B, F // F_BLK)
    gating, mask = pl.pallas_call(
        _gate_body,
        grid=grid,
        in_specs=[
            pl.BlockSpec((1, F_BLK, C), lambda b, f: (b, f, 0)),
            pl.BlockSpec((E, F_BLK), lambda b, f: (0, f)),
            pl.BlockSpec((E,), lambda b, f: (0,)),
        ],
        out_specs=[
            pl.BlockSpec((B, E), lambda b, f: (0, 0)),
            pl.BlockSpec((B, E), lambda b, f: (0, 0)),
        ],
        out_shape=[
            jax.ShapeDtypeStruct((B, E), jnp.float32),
            jax.ShapeDtypeStruct((B, E), jnp.float32),
        ],
        scratch_shapes=[pltpu.VMEM((B, E), jnp.float32)],
    )(x, W, b)
    return gating, mask
